# baseline (speedup 1.0000x reference)
import functools

import jax
import jax.numpy as jnp
from jax import lax
from jax.experimental import pallas as pl
from jax.experimental.pallas import tpu as pltpu
from jax.experimental.pallas import tpu_sc as plsc

NC = 2
NS = 16
NW = NC * NS
IDXW = 128


def _sc_gather(coeffs, table):
    batch, seq = coeffs.shape
    n_rows, d = table.shape
    b = batch * seq
    rows_w = batch // NW
    rows_step = 16
    chunk = rows_step * seq
    steps = rows_w // rows_step
    gathers = chunk // IDXW
    assert batch % NW == 0 and rows_w % rows_step == 0 and chunk % IDXW == 0

    mesh = plsc.VectorSubcoreMesh(core_axis_name="c", subcore_axis_name="s")

    @functools.partial(
        pl.kernel,
        mesh=mesh,
        out_type=jax.ShapeDtypeStruct((b, 128), jnp.float32),
        scratch_types=[
            pltpu.VMEM((chunk,), jnp.int32),
            pltpu.VMEM((chunk, d), jnp.float32),
            pltpu.SemaphoreType.DMA,
            pltpu.SemaphoreType.DMA,
            pltpu.SemaphoreType.DMA,
        ],
        compiler_params=pltpu.CompilerParams(use_tc_tiling_on_sc=False),
    )
    def gather_kernel(table_hbm, coeffs_hbm, out_hbm, idx_v, rows_v,
                      sem_idx, sem_g, sem_st):
        wid = lax.axis_index("s") * NC + lax.axis_index("c")
        row0 = wid * rows_w
        out0 = wid * rows_w * seq

        def body(s, carry):
            r0 = row0 + s * rows_step
            idx_copies = []
            for r in range(rows_step):
                idx_copies.append(pltpu.make_async_copy(
                    coeffs_hbm.at[r0 + r],
                    idx_v.at[pl.ds(r * seq, seq)],
                    sem_idx))
            for c in idx_copies:
                c.start()
            for c in idx_copies:
                c.wait()

            def fire(j, cc):
                pltpu.make_async_copy(
                    table_hbm.at[idx_v.at[pl.ds(j * IDXW, IDXW)]],
                    rows_v.at[pl.ds(j * IDXW, IDXW)],
                    sem_g).start()
                return cc

            lax.fori_loop(0, gathers, fire, 0)

            def drain(j, cc):
                pltpu.make_async_copy(
                    table_hbm.at[idx_v.at[pl.ds(0, IDXW)]],
                    rows_v.at[pl.ds(0, IDXW)],
                    sem_g).wait()
                return cc

            lax.fori_loop(0, gathers, drain, 0)

            st = pltpu.make_async_copy(
                rows_v,
                out_hbm.at[pl.ds(out0 + s * chunk, chunk), pl.ds(0, d)],
                sem_st)
            st.start()
            st.wait()
            return carry

        lax.fori_loop(0, steps, body, 0)

    return gather_kernel(table, coeffs)


def kernel(coeffs, table):
    batch, seq = coeffs.shape
    d = table.shape[1]
    out128 = _sc_gather(coeffs, table)
    return out128[:, :d].reshape(batch, seq, d)
